# Initial kernel scaffold; baseline (speedup 1.0000x reference)
import jax
import jax.numpy as jnp
from jax.experimental import pallas as pl


def kernel(x, W, b):
    raise NotImplementedError("write your pallas kernel here")


# speedup vs baseline: 71.0008x; 71.0008x over previous
import functools

import jax
import jax.numpy as jnp
import numpy as np
from jax import lax
from jax.experimental import pallas as pl
from jax.experimental.pallas import tpu as pltpu
from jax.experimental.pallas import tpu_sc as plsc

_H = 48
_W = 48
_T = _H * _W
_ROWS = 128
_TT = 768
_EPS = np.float32(1e-8)
_DENOM = np.float32(2.0 * 0.1 ** 2)
_NEG1 = np.float32(-1.0)


def _topk_body(cxc_ref, cyc_ref, cxr_ref, cyr_ref, idx_ref):
    i = pl.program_id(0)
    dx = cxc_ref[...] - cxr_ref[...]
    dy = cyc_ref[...] - cyr_ref[...]
    s = dx * dx + dy * dy
    dist = jnp.sqrt(s + _EPS)
    sim = jnp.exp(-(dist * dist) / _DENOM)
    col = lax.broadcasted_iota(jnp.int32, (_ROWS, _T), 1)
    row_tok = i * _ROWS + lax.broadcasted_iota(jnp.int32, (_ROWS, _T), 0)
    sim = jnp.where(col == row_tok, _NEG1, sim)
    m1 = jnp.max(sim, axis=1, keepdims=True)
    c1 = jnp.min(jnp.where(sim == m1, col, _T), axis=1, keepdims=True)
    sim = jnp.where(col == c1, _NEG1, sim)
    m2 = jnp.max(sim, axis=1, keepdims=True)
    c2 = jnp.min(jnp.where(sim == m2, col, _T), axis=1, keepdims=True)
    idx_ref[0, :, 0:1] = c1
    idx_ref[0, :, 1:2] = c2


def _neighbor_idx(cx, cy):
    grid = (_T // _ROWS,)
    cxc = cx.reshape(_T, 1)
    cyc = cy.reshape(_T, 1)
    cxr = cx.reshape(1, _T)
    cyr = cy.reshape(1, _T)
    out = pl.pallas_call(
        _topk_body,
        grid=grid,
        in_specs=[
            pl.BlockSpec((_ROWS, 1), lambda i: (i, 0)),
            pl.BlockSpec((_ROWS, 1), lambda i: (i, 0)),
            pl.BlockSpec((1, _T), lambda i: (0, 0)),
            pl.BlockSpec((1, _T), lambda i: (0, 0)),
        ],
        out_specs=pl.BlockSpec((1, _ROWS, 128), lambda i: (i, 0, 0)),
        out_shape=jax.ShapeDtypeStruct((_T // _ROWS, _ROWS, 128), jnp.int32),
    )(cxc, cyc, cxr, cyr)
    idx1 = out[:, :, 0].reshape(_T)
    idx2 = out[:, :, 1].reshape(_T)
    return idx1, idx2


def _make_sc_gather(n_rows, n_ch, chunk, n_chunks_per_worker):
    info = plsc.get_sparse_core_info()
    nc, ns = info.num_cores, info.num_subcores
    rpw = n_rows // (nc * ns)
    mesh = plsc.VectorSubcoreMesh(core_axis_name="c", subcore_axis_name="s")

    @functools.partial(
        pl.kernel,
        mesh=mesh,
        out_type=jax.ShapeDtypeStruct((n_rows, n_ch), jnp.float32),
        scratch_types=[
            pltpu.VMEM((n_chunks_per_worker, chunk), jnp.int32),
            pltpu.VMEM((rpw, n_ch), jnp.float32),
            pltpu.SemaphoreType.DMA,
        ],
    )
    def sc_gather(table_hbm, idx_hbm, out_hbm, idx_v, rows_v, sem):
        wid = lax.axis_index("s") * nc + lax.axis_index("c")
        base = wid * rpw
        pltpu.sync_copy(idx_hbm.at[wid], idx_v)
        for j in range(n_chunks_per_worker):
            pltpu.async_copy(table_hbm.at[idx_v.at[j]],
                             rows_v.at[pl.ds(j * chunk, chunk)], sem).wait()
        pltpu.sync_copy(rows_v, out_hbm.at[pl.ds(base, rpw)])

    return sc_gather


def _conv_body(featT_ref, prime_ref, w_ref, bias_ref, out_ref):
    f0 = featT_ref[0]
    p1 = prime_ref[0, 0]
    p2 = prime_ref[0, 1]
    dn = (((1,), (1,)), ((), ()))
    acc = lax.dot_general(w_ref[0], f0, dn, preferred_element_type=jnp.float32)
    acc = acc + lax.dot_general(w_ref[1], p1, dn,
                                preferred_element_type=jnp.float32)
    acc = acc + lax.dot_general(w_ref[2], p2, dn,
                                preferred_element_type=jnp.float32)
    out_ref[0] = acc + bias_ref[...]


def kernel(x, W, b):
    B, C, H, Wd = x.shape
    O = W.shape[0]
    T = H * Wd

    yv = jnp.linspace(-1.0, 1.0, H)
    xv = jnp.linspace(-1.0, 1.0, Wd)
    yg, xg = jnp.meshgrid(yv, xv, indexing="ij")
    cx = xg.reshape(T)
    cy = yg.reshape(T)

    idx1, idx2 = _neighbor_idx(cx, cy)

    feat = x.reshape(B, C, T)
    featT = jnp.swapaxes(feat, 1, 2)
    table = featT.reshape(B * T, C)

    off = (jnp.arange(B, dtype=jnp.int32) * T)[:, None, None]
    idx_kt = jnp.stack([idx1, idx2], axis=0)[None]
    n_rows = B * 2 * T
    chunk = 96
    n_workers = 32
    n_chunks_per_worker = n_rows // (n_workers * chunk)
    gidx = (idx_kt + off).reshape(n_workers, n_chunks_per_worker, chunk)

    sc_gather = _make_sc_gather(n_rows, C, chunk, n_chunks_per_worker)
    prime = sc_gather(table, gidx).reshape(B, 2, T, C)

    Wk = jnp.moveaxis(W, 2, 0)
    bias_col = b.reshape(O, 1)
    out = pl.pallas_call(
        _conv_body,
        grid=(B, T // _TT),
        in_specs=[
            pl.BlockSpec((1, _TT, C), lambda bi, j: (bi, j, 0)),
            pl.BlockSpec((1, 2, _TT, C), lambda bi, j: (bi, 0, j, 0)),
            pl.BlockSpec((3, O, C), lambda bi, j: (0, 0, 0)),
            pl.BlockSpec((O, 1), lambda bi, j: (0, 0)),
        ],
        out_specs=pl.BlockSpec((1, O, _TT), lambda bi, j: (bi, 0, j)),
        out_shape=jax.ShapeDtypeStruct((B, O, T), jnp.float32),
    )(featT, prime, Wk, bias_col)
    return out.reshape(B, O, H, Wd)


# speedup vs baseline: 118.2222x; 1.6651x over previous
import functools

import jax
import jax.numpy as jnp
import numpy as np
from jax import lax
from jax.experimental import pallas as pl
from jax.experimental.pallas import tpu as pltpu
from jax.experimental.pallas import tpu_sc as plsc

_H = 48
_W = 48
_T = _H * _W
_ROWS = 128
_TT = 768
_EPS = np.float32(1e-8)
_DENOM = np.float32(2.0 * 0.1 ** 2)
_NEG1 = np.float32(-1.0)


_SUB = _T // 128
_OFFS = (-_W, -1, 1, _W)


def _topk_body(cx_ref, cy_ref, cxs_ref, cys_ref, val_ref, idx_ref):
    cx = cx_ref[...]
    cy = cy_ref[...]
    tok = (lax.broadcasted_iota(jnp.int32, (_SUB, 128), 0) * 128
           + lax.broadcasted_iota(jnp.int32, (_SUB, 128), 1))
    sims = []
    cidx = []
    for j in range(4):
        dx = cx - cxs_ref[j]
        dy = cy - cys_ref[j]
        s = dx * dx + dy * dy
        dist = jnp.sqrt(s + _EPS)
        sim = jnp.exp(-(dist * dist) / _DENOM)
        sims.append(jnp.where(val_ref[j] > 0, sim, _NEG1))
        cidx.append(tok + _OFFS[j])
    m1 = jnp.maximum(jnp.maximum(sims[0], sims[1]),
                     jnp.maximum(sims[2], sims[3]))
    idx1 = jnp.where(sims[0] == m1, cidx[0],
                     jnp.where(sims[1] == m1, cidx[1],
                               jnp.where(sims[2] == m1, cidx[2], cidx[3])))
    sims = [jnp.where(cidx[j] == idx1, _NEG1, sims[j]) for j in range(4)]
    m2 = jnp.maximum(jnp.maximum(sims[0], sims[1]),
                     jnp.maximum(sims[2], sims[3]))
    idx2 = jnp.where(sims[0] == m2, cidx[0],
                     jnp.where(sims[1] == m2, cidx[1],
                               jnp.where(sims[2] == m2, cidx[2], cidx[3])))
    idx_ref[0] = idx1
    idx_ref[1] = idx2


def _neighbor_idx(cx, cy):
    xg = cx.reshape(_H, _W)
    yg = cy.reshape(_H, _W)
    cxs = jnp.stack([jnp.roll(xg, 1, 0), jnp.roll(xg, 1, 1),
                     jnp.roll(xg, -1, 1), jnp.roll(xg, -1, 0)])
    cys = jnp.stack([jnp.roll(yg, 1, 0), jnp.roll(yg, 1, 1),
                     jnp.roll(yg, -1, 1), jnp.roll(yg, -1, 0)])
    yy = jnp.arange(_H, dtype=jnp.float32)[:, None] * jnp.ones((1, _W), jnp.float32)
    xx = jnp.ones((_H, 1), jnp.float32) * jnp.arange(_W, dtype=jnp.float32)[None]
    val = jnp.stack([(yy > 0), (xx > 0), (xx < _W - 1), (yy < _H - 1)])
    out = pl.pallas_call(
        _topk_body,
        out_shape=jax.ShapeDtypeStruct((2, _SUB, 128), jnp.int32),
    )(cx.reshape(_SUB, 128), cy.reshape(_SUB, 128),
      cxs.reshape(4, _SUB, 128), cys.reshape(4, _SUB, 128),
      val.astype(jnp.float32).reshape(4, _SUB, 128))
    return out[0].reshape(_T), out[1].reshape(_T)


def _make_sc_gather(n_rows, n_ch, chunk, n_chunks_per_worker):
    info = plsc.get_sparse_core_info()
    nc, ns = info.num_cores, info.num_subcores
    rpw = n_rows // (nc * ns)
    mesh = plsc.VectorSubcoreMesh(core_axis_name="c", subcore_axis_name="s")

    @functools.partial(
        pl.kernel,
        mesh=mesh,
        out_type=jax.ShapeDtypeStruct((n_rows, n_ch), jnp.float32),
        scratch_types=[
            pltpu.VMEM((n_chunks_per_worker, chunk), jnp.int32),
            pltpu.VMEM((rpw, n_ch), jnp.float32),
            pltpu.SemaphoreType.DMA,
        ],
    )
    def sc_gather(table_hbm, idx_hbm, out_hbm, idx_v, rows_v, sem):
        wid = lax.axis_index("s") * nc + lax.axis_index("c")
        base = wid * rpw
        pltpu.sync_copy(idx_hbm.at[wid], idx_v)
        copies = [
            pltpu.async_copy(table_hbm.at[idx_v.at[j]],
                             rows_v.at[pl.ds(j * chunk, chunk)], sem)
            for j in range(n_chunks_per_worker)
        ]
        for c in copies:
            c.wait()
        pltpu.sync_copy(rows_v, out_hbm.at[pl.ds(base, rpw)])

    return sc_gather


def _conv_body(featT_ref, prime_ref, w_ref, bias_ref, out_ref):
    f0 = featT_ref[0]
    p1 = prime_ref[0, 0]
    p2 = prime_ref[0, 1]
    dn = (((1,), (1,)), ((), ()))
    acc = lax.dot_general(w_ref[0], f0, dn, preferred_element_type=jnp.float32)
    acc = acc + lax.dot_general(w_ref[1], p1, dn,
                                preferred_element_type=jnp.float32)
    acc = acc + lax.dot_general(w_ref[2], p2, dn,
                                preferred_element_type=jnp.float32)
    out_ref[0] = acc + bias_ref[...]


def kernel(x, W, b):
    B, C, H, Wd = x.shape
    O = W.shape[0]
    T = H * Wd

    yv = jnp.linspace(-1.0, 1.0, H)
    xv = jnp.linspace(-1.0, 1.0, Wd)
    yg, xg = jnp.meshgrid(yv, xv, indexing="ij")
    cx = xg.reshape(T)
    cy = yg.reshape(T)

    idx1, idx2 = _neighbor_idx(cx, cy)

    feat = x.reshape(B, C, T)
    featT = jnp.swapaxes(feat, 1, 2)
    table = featT.reshape(B * T, C)

    off = (jnp.arange(B, dtype=jnp.int32) * T)[:, None, None]
    idx_kt = jnp.stack([idx1, idx2], axis=0)[None]
    n_rows = B * 2 * T
    chunk = 96
    n_workers = 32
    n_chunks_per_worker = n_rows // (n_workers * chunk)
    gidx = (idx_kt + off).reshape(n_workers, n_chunks_per_worker, chunk)

    sc_gather = _make_sc_gather(n_rows, C, chunk, n_chunks_per_worker)
    prime = sc_gather(table, gidx).reshape(B, 2, T, C)

    Wk = jnp.moveaxis(W, 2, 0)
    bias_col = b.reshape(O, 1)
    out = pl.pallas_call(
        _conv_body,
        grid=(B, T // _TT),
        in_specs=[
            pl.BlockSpec((1, _TT, C), lambda bi, j: (bi, j, 0)),
            pl.BlockSpec((1, 2, _TT, C), lambda bi, j: (bi, 0, j, 0)),
            pl.BlockSpec((3, O, C), lambda bi, j: (0, 0, 0)),
            pl.BlockSpec((O, 1), lambda bi, j: (0, 0)),
        ],
        out_specs=pl.BlockSpec((1, O, _TT), lambda bi, j: (bi, 0, j)),
        out_shape=jax.ShapeDtypeStruct((B, O, T), jnp.float32),
    )(featT, prime, Wk, bias_col)
    return out.reshape(B, O, H, Wd)
